# baseline (speedup 1.0000x reference)
import functools

import jax
import jax.numpy as jnp
from jax import lax
from jax.experimental import pallas as pl
from jax.experimental.pallas import tpu as pltpu
from jax.experimental.pallas import tpu_sc as plsc

N_SRC = 100000
N_TGT = 10000
N_EDGE = 1600000
SUP = 512
GS = 256
NG = SUP // GS
NCH = N_EDGE // SUP
CPT = -(-NCH // 32)
STR = 632
BZ = 4168


def _embed_body(sf_ref, xn_ref, emb_ref, hw_ref, z_ref):
    w2 = jnp.dot(emb_ref[...], hw_ref[...], preferred_element_type=jnp.float32)
    wbig = jnp.concatenate(
        [jnp.pad(w2, ((0, 0), (16 * j, 112 - 16 * j))) for j in range(8)],
        axis=0)
    z = jnp.dot(sf_ref[...], wbig, preferred_element_type=jnp.float32)
    i0 = lax.broadcasted_iota(jnp.int32, (8, 128), 0)
    i1 = lax.broadcasted_iota(jnp.int32, (8, 128), 1)
    sel = (i1 // 16 == i0).astype(jnp.float32)
    xnp = jnp.dot(xn_ref[...], sel, preferred_element_type=jnp.float32)
    z_ref[...] = z / xnp


def _embed_call(source_feat, x_norm, embed, hgcn_weight):
    sf8 = source_feat.reshape(N_SRC // 8, 1024)
    xn8 = x_norm.reshape(N_SRC // 8, 8)
    grid = (-(-(N_SRC // 8) // BZ),)
    return pl.pallas_call(
        _embed_body,
        grid=grid,
        in_specs=[
            pl.BlockSpec((BZ, 1024), lambda i: (i, 0)),
            pl.BlockSpec((BZ, 8), lambda i: (i, 0)),
            pl.BlockSpec((128, 32), lambda i: (0, 0)),
            pl.BlockSpec((32, 16), lambda i: (0, 0)),
        ],
        out_specs=pl.BlockSpec((BZ, 128), lambda i: (i, 0)),
        out_shape=jax.ShapeDtypeStruct((N_SRC // 8, 128), jnp.float32),
    )(sf8, xn8, embed, hgcn_weight)


_MESH = plsc.VectorSubcoreMesh(
    core_axis_name="c", subcore_axis_name="s", num_cores=2, num_subcores=16
)


def _sc_body(z_hbm, ei_hbm, zer_hbm, out_hbm,
             sbuf, tbuf, rows, acc, isem, gsem, asem):
    c = lax.axis_index("c")
    s = lax.axis_index("s")

    zb = jnp.minimum(s * STR, N_TGT - STR)
    pltpu.sync_copy(zer_hbm.at[pl.ds(zb, STR), :], acc.at[pl.ds(zb, STR), :])
    plsc.subcore_barrier()

    w = c * 16 + s
    k0 = w * CPT
    k1 = jnp.minimum(k0 + CPT, NCH)
    nit = jnp.maximum(k1 - k0, 0)

    def start_idx(k, b):
        pltpu.async_copy(ei_hbm.at[0, k], sbuf.at[b], isem)
        pltpu.async_copy(ei_hbm.at[1, k], tbuf.at[b], isem)

    def wait_idx(b):
        pltpu.make_async_copy(ei_hbm.at[0, 0], sbuf.at[b], isem).wait()
        pltpu.make_async_copy(ei_hbm.at[1, 0], tbuf.at[b], isem).wait()

    def fire_gathers(b):
        for j in range(NG):
            pltpu.async_copy(z_hbm.at[sbuf.at[b, j]],
                             rows.at[b, pl.ds(j * GS, GS), :], gsem)

    def wait_gathers(b):
        for j in range(NG):
            pltpu.make_async_copy(z_hbm.at[sbuf.at[b, j]],
                                  rows.at[b, pl.ds(j * GS, GS), :],
                                  gsem).wait()

    def fire_adds(b):
        for j in range(NG):
            pltpu.async_copy(rows.at[b, pl.ds(j * GS, GS), :],
                             acc.at[tbuf.at[b, j]], asem, add=True)

    def wait_adds(b):
        for j in range(NG):
            pltpu.make_async_copy(rows.at[b, pl.ds(j * GS, GS), :],
                                  acc.at[tbuf.at[b, j]], asem).wait()

    @pl.when(nit > 0)
    def _prime():
        start_idx(k0, 0)

    @pl.loop(0, nit)
    def _chunk(i):
        b = lax.rem(i, 2)
        nb = 1 - b
        wait_idx(b)
        fire_gathers(b)

        @pl.when(i > 0)
        def _():
            wait_adds(nb)

        @pl.when(i + 1 < nit)
        def _():
            start_idx(k0 + i + 1, nb)

        wait_gathers(b)
        fire_adds(b)

    @pl.when(nit > 0)
    def _drain():
        wait_adds(lax.rem(nit - 1, 2))

    plsc.subcore_barrier()
    pltpu.sync_copy(acc.at[pl.ds(zb, STR), :], out_hbm.at[c, pl.ds(zb, STR), :])


_sc_call = functools.partial(
    pl.kernel,
    out_type=jax.ShapeDtypeStruct((2, N_TGT, 16), jnp.float32),
    mesh=_MESH,
    scratch_types=[
        pltpu.VMEM((2, NG, GS), jnp.int32),
        pltpu.VMEM((2, NG, GS), jnp.int32),
        pltpu.VMEM((2, SUP, 16), jnp.float32),
        pltpu.VMEM_SHARED((N_TGT, 16), jnp.float32),
        pltpu.SemaphoreType.DMA,
        pltpu.SemaphoreType.DMA,
        pltpu.SemaphoreType.DMA,
    ],
    compiler_params=pltpu.CompilerParams(use_tc_tiling_on_sc=False),
)(_sc_body)


def _fin_body(p_ref, rl_ref, out_ref):
    ssum = p_ref[0] + p_ref[1]
    rl = rl_ref[...]
    deg = (rl[:, 1:2] - rl[:, 0:1]).astype(jnp.float32)
    out_ref[...] = ssum / jnp.maximum(deg, 1.0)


def _fin_call(partials, range_list):
    return pl.pallas_call(
        _fin_body,
        out_shape=jax.ShapeDtypeStruct((N_TGT, 16), jnp.float32),
    )(partials, range_list)


def kernel(source_feat, edge_index, range_list, x_norm, embed, hgcn_weight):
    ei = edge_index.astype(jnp.int32).reshape(2, -1, NG, GS)
    z = _embed_call(source_feat, x_norm, embed, hgcn_weight)
    z = z.reshape(N_SRC, 16)
    zer = jnp.zeros((N_TGT, 16), jnp.float32)
    partials = _sc_call(z, ei, zer)
    return _fin_call(partials, range_list.astype(jnp.int32))


# speedup vs baseline: 1.0071x; 1.0071x over previous
import functools

import jax
import jax.numpy as jnp
from jax import lax
from jax.experimental import pallas as pl
from jax.experimental.pallas import tpu as pltpu
from jax.experimental.pallas import tpu_sc as plsc

N_SRC = 100000
N_TGT = 10000
N_EDGE = 1600000
SUP = 512
GS = 256
NG = SUP // GS
NCH = N_EDGE // SUP
CPT = -(-NCH // 32)
STR = 632
ZSTR = 6256
BZ = 1664


def _embed_body(sf_ref, xn_ref, emb_ref, hw_ref, z_ref):
    w2 = jnp.dot(emb_ref[...], hw_ref[...], preferred_element_type=jnp.float32)
    wbig = jnp.concatenate(
        [jnp.pad(w2, ((0, 0), (16 * j, 112 - 16 * j))) for j in range(8)],
        axis=0)
    z = jnp.dot(sf_ref[...], wbig, preferred_element_type=jnp.float32)
    i0 = lax.broadcasted_iota(jnp.int32, (8, 128), 0)
    i1 = lax.broadcasted_iota(jnp.int32, (8, 128), 1)
    sel = (i1 // 16 == i0).astype(jnp.float32)
    xnp = jnp.dot(xn_ref[...], sel, preferred_element_type=jnp.float32)
    z_ref[...] = z / xnp


def _embed_call(source_feat, x_norm, embed, hgcn_weight):
    sf8 = source_feat.reshape(N_SRC // 8, 1024)
    xn8 = x_norm.reshape(N_SRC // 8, 8)
    grid = (-(-(N_SRC // 8) // BZ),)
    return pl.pallas_call(
        _embed_body,
        grid=grid,
        in_specs=[
            pl.BlockSpec((BZ, 1024), lambda i: (i, 0)),
            pl.BlockSpec((BZ, 8), lambda i: (i, 0)),
            pl.BlockSpec((128, 32), lambda i: (0, 0)),
            pl.BlockSpec((32, 16), lambda i: (0, 0)),
        ],
        out_specs=pl.BlockSpec((BZ, 128), lambda i: (i, 0)),
        out_shape=jax.ShapeDtypeStruct((N_SRC // 8, 128), jnp.float32),
    )(sf8, xn8, embed, hgcn_weight)


_MESH = plsc.VectorSubcoreMesh(
    core_axis_name="c", subcore_axis_name="s", num_cores=2, num_subcores=16
)


def _sc_body(z_hbm, ei_hbm, zer_hbm, out_hbm,
             sbuf, tbuf, rows, zsp, acc, isem, gsem, asem):
    c = lax.axis_index("c")
    s = lax.axis_index("s")

    zb = jnp.minimum(s * STR, N_TGT - STR)
    pltpu.sync_copy(zer_hbm.at[pl.ds(zb, STR), :], acc.at[pl.ds(zb, STR), :])
    zs = jnp.minimum(s * ZSTR, N_SRC - ZSTR)
    pltpu.sync_copy(z_hbm.at[pl.ds(zs, ZSTR), :], zsp.at[pl.ds(zs, ZSTR), :])
    plsc.subcore_barrier()

    w = c * 16 + s
    k0 = w * CPT
    k1 = jnp.minimum(k0 + CPT, NCH)
    nit = jnp.maximum(k1 - k0, 0)

    def start_idx(k, b):
        pltpu.async_copy(ei_hbm.at[0, k], sbuf.at[b], isem)
        pltpu.async_copy(ei_hbm.at[1, k], tbuf.at[b], isem)

    def wait_idx(b):
        pltpu.make_async_copy(ei_hbm.at[0, 0], sbuf.at[b], isem).wait()
        pltpu.make_async_copy(ei_hbm.at[1, 0], tbuf.at[b], isem).wait()

    def fire_gathers(b):
        for j in range(NG):
            pltpu.async_copy(zsp.at[sbuf.at[b, j]],
                             rows.at[b, pl.ds(j * GS, GS), :], gsem)

    def wait_gathers(b):
        for j in range(NG):
            pltpu.make_async_copy(zsp.at[sbuf.at[b, j]],
                                  rows.at[b, pl.ds(j * GS, GS), :],
                                  gsem).wait()

    def fire_adds(b):
        for j in range(NG):
            pltpu.async_copy(rows.at[b, pl.ds(j * GS, GS), :],
                             acc.at[tbuf.at[b, j]], asem, add=True)

    def wait_adds(b):
        for j in range(NG):
            pltpu.make_async_copy(rows.at[b, pl.ds(j * GS, GS), :],
                                  acc.at[tbuf.at[b, j]], asem).wait()

    @pl.when(nit > 0)
    def _prime():
        start_idx(k0, 0)

    @pl.loop(0, nit)
    def _chunk(i):
        b = lax.rem(i, 2)
        nb = 1 - b
        wait_idx(b)
        fire_gathers(b)

        @pl.when(i > 0)
        def _():
            wait_adds(nb)

        @pl.when(i + 1 < nit)
        def _():
            start_idx(k0 + i + 1, nb)

        wait_gathers(b)
        fire_adds(b)

    @pl.when(nit > 0)
    def _drain():
        wait_adds(lax.rem(nit - 1, 2))

    plsc.subcore_barrier()
    pltpu.sync_copy(acc.at[pl.ds(zb, STR), :], out_hbm.at[c, pl.ds(zb, STR), :])


_sc_call = functools.partial(
    pl.kernel,
    out_type=jax.ShapeDtypeStruct((2, N_TGT, 16), jnp.float32),
    mesh=_MESH,
    scratch_types=[
        pltpu.VMEM((2, NG, GS), jnp.int32),
        pltpu.VMEM((2, NG, GS), jnp.int32),
        pltpu.VMEM((2, SUP, 16), jnp.float32),
        pltpu.VMEM_SHARED((N_SRC, 16), jnp.float32),
        pltpu.VMEM_SHARED((N_TGT, 16), jnp.float32),
        pltpu.SemaphoreType.DMA,
        pltpu.SemaphoreType.DMA,
        pltpu.SemaphoreType.DMA,
    ],
    compiler_params=pltpu.CompilerParams(use_tc_tiling_on_sc=False),
)(_sc_body)


def _fin_body(p_ref, rl_ref, out_ref):
    ssum = p_ref[0] + p_ref[1]
    rl = rl_ref[...]
    deg = (rl[:, 1:2] - rl[:, 0:1]).astype(jnp.float32)
    out_ref[...] = ssum / jnp.maximum(deg, 1.0)


def _fin_call(partials, range_list):
    return pl.pallas_call(
        _fin_body,
        out_shape=jax.ShapeDtypeStruct((N_TGT, 16), jnp.float32),
    )(partials, range_list)


def kernel(source_feat, edge_index, range_list, x_norm, embed, hgcn_weight):
    ei = edge_index.astype(jnp.int32).reshape(2, -1, NG, GS)
    z = _embed_call(source_feat, x_norm, embed, hgcn_weight)
    z = z.reshape(N_SRC, 16)
    zer = jnp.zeros((N_TGT, 16), jnp.float32)
    partials = _sc_call(z, ei, zer)
    return _fin_call(partials, range_list.astype(jnp.int32))


# speedup vs baseline: 1.0132x; 1.0060x over previous
import functools

import jax
import jax.numpy as jnp
from jax import lax
from jax.experimental import pallas as pl
from jax.experimental.pallas import tpu as pltpu
from jax.experimental.pallas import tpu_sc as plsc

N_SRC = 100000
N_TGT = 10000
N_EDGE = 1600000
SUP = 512
GS = 256
NG = SUP // GS
NCH = N_EDGE // SUP
CPT = -(-NCH // 32)
STR = 632
BZ = 1664


def _embed_body(sf_ref, xn_ref, emb_ref, hw_ref, z_ref):
    w2 = jnp.dot(emb_ref[...], hw_ref[...], preferred_element_type=jnp.float32)
    wbig = jnp.concatenate(
        [jnp.pad(w2, ((0, 0), (16 * j, 112 - 16 * j))) for j in range(8)],
        axis=0)
    z = jnp.dot(sf_ref[...], wbig, preferred_element_type=jnp.float32)
    i0 = lax.broadcasted_iota(jnp.int32, (8, 128), 0)
    i1 = lax.broadcasted_iota(jnp.int32, (8, 128), 1)
    sel = (i1 // 16 == i0).astype(jnp.float32)
    xnp = jnp.dot(xn_ref[...], sel, preferred_element_type=jnp.float32)
    z_ref[...] = z / xnp


def _embed_call(source_feat, x_norm, embed, hgcn_weight):
    sf8 = source_feat.reshape(N_SRC // 8, 1024)
    xn8 = x_norm.reshape(N_SRC // 8, 8)
    grid = (-(-(N_SRC // 8) // BZ),)
    return pl.pallas_call(
        _embed_body,
        grid=grid,
        in_specs=[
            pl.BlockSpec((BZ, 1024), lambda i: (i, 0)),
            pl.BlockSpec((BZ, 8), lambda i: (i, 0)),
            pl.BlockSpec((128, 32), lambda i: (0, 0)),
            pl.BlockSpec((32, 16), lambda i: (0, 0)),
        ],
        out_specs=pl.BlockSpec((BZ, 128), lambda i: (i, 0)),
        out_shape=jax.ShapeDtypeStruct((N_SRC // 8, 128), jnp.float32),
    )(sf8, xn8, embed, hgcn_weight)


_MESH = plsc.VectorSubcoreMesh(
    core_axis_name="c", subcore_axis_name="s", num_cores=2, num_subcores=16
)


def _sc_body(z_hbm, ei_hbm, zer_hbm, out_hbm,
             sbuf, tbuf, rows, acc, isem, gsem, asem):
    c = lax.axis_index("c")
    s = lax.axis_index("s")

    zb = jnp.minimum(s * STR, N_TGT - STR)
    pltpu.sync_copy(zer_hbm.at[pl.ds(zb, STR), :], acc.at[pl.ds(zb, STR), :])
    plsc.subcore_barrier()

    w = c * 16 + s
    k0 = w * CPT
    k1 = jnp.minimum(k0 + CPT, NCH)
    nit = jnp.maximum(k1 - k0, 0)

    def start_idx(k, b):
        pltpu.async_copy(ei_hbm.at[0, k], sbuf.at[b], isem)
        pltpu.async_copy(ei_hbm.at[1, k], tbuf.at[b], isem)

    def wait_idx(b):
        pltpu.make_async_copy(ei_hbm.at[0, 0], sbuf.at[b], isem).wait()
        pltpu.make_async_copy(ei_hbm.at[1, 0], tbuf.at[b], isem).wait()

    def fire_gathers(b):
        for j in range(NG):
            pltpu.async_copy(z_hbm.at[sbuf.at[b, j]],
                             rows.at[b, pl.ds(j * GS, GS), :], gsem)

    def wait_gathers(b):
        for j in range(NG):
            pltpu.make_async_copy(z_hbm.at[sbuf.at[b, j]],
                                  rows.at[b, pl.ds(j * GS, GS), :],
                                  gsem).wait()

    def fire_adds(b):
        for j in range(NG):
            pltpu.async_copy(rows.at[b, pl.ds(j * GS, GS), :],
                             acc.at[tbuf.at[b, j]], asem, add=True)

    def wait_adds(b):
        for j in range(NG):
            pltpu.make_async_copy(rows.at[b, pl.ds(j * GS, GS), :],
                                  acc.at[tbuf.at[b, j]], asem).wait()

    @pl.when(nit > 0)
    def _prime():
        start_idx(k0, 0)

    @pl.loop(0, nit)
    def _chunk(i):
        b = lax.rem(i, 3)

        @pl.when(i > 1)
        def _():
            wait_adds(lax.rem(i - 2, 3))

        wait_idx(b)
        fire_gathers(b)

        @pl.when(i + 1 < nit)
        def _():
            start_idx(k0 + i + 1, lax.rem(i + 1, 3))

        wait_gathers(b)
        fire_adds(b)

    @pl.when(nit > 1)
    def _drain2():
        wait_adds(lax.rem(nit - 2, 3))

    @pl.when(nit > 0)
    def _drain1():
        wait_adds(lax.rem(nit - 1, 3))

    plsc.subcore_barrier()
    pltpu.sync_copy(acc.at[pl.ds(zb, STR), :], out_hbm.at[c, pl.ds(zb, STR), :])


_sc_call = functools.partial(
    pl.kernel,
    out_type=jax.ShapeDtypeStruct((2, N_TGT, 16), jnp.float32),
    mesh=_MESH,
    scratch_types=[
        pltpu.VMEM((3, NG, GS), jnp.int32),
        pltpu.VMEM((3, NG, GS), jnp.int32),
        pltpu.VMEM((3, SUP, 16), jnp.float32),
        pltpu.VMEM_SHARED((N_TGT, 16), jnp.float32),
        pltpu.SemaphoreType.DMA,
        pltpu.SemaphoreType.DMA,
        pltpu.SemaphoreType.DMA,
    ],
    compiler_params=pltpu.CompilerParams(use_tc_tiling_on_sc=False),
)(_sc_body)


def _fin_body(p_ref, rl_ref, out_ref):
    ssum = p_ref[0] + p_ref[1]
    rl = rl_ref[...]
    deg = (rl[:, 1:2] - rl[:, 0:1]).astype(jnp.float32)
    out_ref[...] = ssum / jnp.maximum(deg, 1.0)


def _fin_call(partials, range_list):
    return pl.pallas_call(
        _fin_body,
        out_shape=jax.ShapeDtypeStruct((N_TGT, 16), jnp.float32),
    )(partials, range_list)


def kernel(source_feat, edge_index, range_list, x_norm, embed, hgcn_weight):
    ei = edge_index.astype(jnp.int32).reshape(2, -1, NG, GS)
    z = _embed_call(source_feat, x_norm, embed, hgcn_weight)
    z = z.reshape(N_SRC, 16)
    zer = jnp.zeros((N_TGT, 16), jnp.float32)
    partials = _sc_call(z, ei, zer)
    return _fin_call(partials, range_list.astype(jnp.int32))
